# baseline (speedup 1.0000x reference)
import functools

import jax
import jax.numpy as jnp
from jax import lax
from jax.experimental import pallas as pl
from jax.experimental.pallas import tpu as pltpu
from jax.experimental.pallas import tpu_sc as plsc

B = 16384
HALF = B // 2
IN_DIM = 768
OUT_DIM = 256
K = 1024
BETA = 0.25
EPS = 1e-5

TILE = 512
GRID = HALF // TILE


def _argmin_dist(r, cb):
    rn = jnp.sum(r ** 2, axis=1, keepdims=True)
    dot = lax.dot_general(r, cb, (((1,), (1,)), ((), ())))
    cbn = jnp.sum(cb ** 2, axis=1)[None, :]
    d = rn - 2.0 * dot + cbn
    return jnp.argmin(d, axis=1).astype(jnp.int32)


def _enc_body(x_ref, g_ref, be_ref, W1_ref, b1_ref, W2_ref, b2_ref, cb_ref,
              enc_ref, idx_ref):
    x = x_ref[...]
    mu = jnp.mean(x, axis=-1, keepdims=True)
    var = jnp.mean((x - mu) ** 2, axis=-1, keepdims=True)
    xn = (x - mu) / jnp.sqrt(var + EPS) * g_ref[...] + be_ref[...]
    h = jnp.maximum(jnp.dot(xn, W1_ref[...]) + b1_ref[...], 0.0)
    enc = jnp.dot(h, W2_ref[...]) + b2_ref[...]
    enc_ref[...] = enc
    idx_ref[...] = _argmin_dist(enc, cb_ref[...])[:, None]


def _level_body(rprev_ref, q_ref, cb_ref, rout_ref, idx_ref, csum_ref):
    i = pl.program_id(0)
    r = rprev_ref[...] - q_ref[...]
    rout_ref[...] = r
    idx_ref[...] = _argmin_dist(r, cb_ref[...])[:, None]
    s = jnp.sum(r * r).reshape(1, 1)

    @pl.when(i == 0)
    def _():
        csum_ref[...] = s

    @pl.when(i > 0)
    def _():
        csum_ref[...] = csum_ref[...] + s


def _final_body(r2_ref, q2_ref, enc_ref, x_ref, Wd1_ref, bd1_ref, Wd2_ref,
                bd2_ref, rec_ref, part_ref, accr, accc):
    i = pl.program_id(0)
    r3 = r2_ref[...] - q2_ref[...]
    fq = enc_ref[...] - r3
    dh = jnp.maximum(jnp.dot(fq, Wd1_ref[...]) + bd1_ref[...], 0.0)
    rec = jnp.dot(dh, Wd2_ref[...]) + bd2_ref[...]
    rec_ref[...] = rec
    diff = rec - x_ref[...]
    rs = jnp.sum(diff * diff).reshape(1, 1)
    cs = jnp.sum(r3 * r3).reshape(1, 1)

    @pl.when(i == 0)
    def _():
        accr[...] = rs
        accc[...] = cs

    @pl.when(i > 0)
    def _():
        accr[...] = accr[...] + rs
        accc[...] = accc[...] + cs

    @pl.when(i == GRID - 1)
    def _():
        part_ref[...] = jnp.concatenate([accr[...], accc[...]], axis=1)


def _loss_body(c0a_ref, c0b_ref, c1a_ref, c1b_ref, pa_ref, pb_ref, loss_ref):
    rl = (pa_ref[0, 0] + pb_ref[0, 0]) / jnp.float32(B * IN_DIM)
    cl0 = (c0a_ref[0, 0] + c0b_ref[0, 0]) / jnp.float32(B * OUT_DIM)
    cl1 = (c1a_ref[0, 0] + c1b_ref[0, 0]) / jnp.float32(B * OUT_DIM)
    cl2 = (pa_ref[0, 1] + pb_ref[0, 1]) / jnp.float32(B * OUT_DIM)
    loss_ref[0, 0] = rl + BETA * (cl0 + cl1 + cl2)


def _row_spec(w):
    return pl.BlockSpec((TILE, w), lambda i: (i, 0))


def _const_spec(h, w):
    return pl.BlockSpec((h, w), lambda i: (0, 0))


_NC = 2
_NS = 16
_NW = _NC * _NS
_ROWS_PER_W = HALF // _NW


def _sc_gather(table, idx):
    mesh = plsc.VectorSubcoreMesh(core_axis_name="c", subcore_axis_name="s")

    @functools.partial(
        pl.kernel,
        mesh=mesh,
        out_type=jax.ShapeDtypeStruct((HALF, OUT_DIM), jnp.float32),
        scratch_types=[
            pltpu.VMEM((_ROWS_PER_W,), jnp.int32),
            pltpu.VMEM((_ROWS_PER_W, OUT_DIM), jnp.float32),
            pltpu.SemaphoreType.DMA,
        ],
    )
    def gk(table_hbm, idx_hbm, out_hbm, idx_v, rows_v, sem):
        wid = lax.axis_index("s") * _NC + lax.axis_index("c")
        base = wid * _ROWS_PER_W
        pltpu.sync_copy(idx_hbm.at[pl.ds(base, _ROWS_PER_W)], idx_v)
        pltpu.async_copy(table_hbm.at[idx_v], rows_v, sem).wait()
        pltpu.sync_copy(rows_v, out_hbm.at[pl.ds(base, _ROWS_PER_W)])

    return gk(table, idx)


def kernel(x, ln_g, ln_b, W1, b1, W2, b2, Wd1, bd1, Wd2, bd2, codebooks):
    g2 = ln_g[None, :]
    be2 = ln_b[None, :]
    b12 = b1[None, :]
    b22 = b2[None, :]
    bd12 = bd1[None, :]
    bd22 = bd2[None, :]
    cb0, cb1, cb2 = codebooks[0], codebooks[1], codebooks[2]

    enc_call = pl.pallas_call(
        _enc_body,
        grid=(GRID,),
        in_specs=[
            _row_spec(IN_DIM),
            _const_spec(1, IN_DIM),
            _const_spec(1, IN_DIM),
            _const_spec(IN_DIM, OUT_DIM),
            _const_spec(1, OUT_DIM),
            _const_spec(OUT_DIM, OUT_DIM),
            _const_spec(1, OUT_DIM),
            _const_spec(K, OUT_DIM),
        ],
        out_specs=[_row_spec(OUT_DIM), _row_spec(1)],
        out_shape=[
            jax.ShapeDtypeStruct((HALF, OUT_DIM), jnp.float32),
            jax.ShapeDtypeStruct((HALF, 1), jnp.int32),
        ],
    )

    level_call = pl.pallas_call(
        _level_body,
        grid=(GRID,),
        in_specs=[_row_spec(OUT_DIM), _row_spec(OUT_DIM), _const_spec(K, OUT_DIM)],
        out_specs=[_row_spec(OUT_DIM), _row_spec(1), _const_spec(1, 1)],
        out_shape=[
            jax.ShapeDtypeStruct((HALF, OUT_DIM), jnp.float32),
            jax.ShapeDtypeStruct((HALF, 1), jnp.int32),
            jax.ShapeDtypeStruct((1, 1), jnp.float32),
        ],
    )

    final_call = pl.pallas_call(
        _final_body,
        grid=(GRID,),
        in_specs=[
            _row_spec(OUT_DIM),
            _row_spec(OUT_DIM),
            _row_spec(OUT_DIM),
            _row_spec(IN_DIM),
            _const_spec(OUT_DIM, IN_DIM),
            _const_spec(1, IN_DIM),
            _const_spec(IN_DIM, IN_DIM),
            _const_spec(1, IN_DIM),
        ],
        out_specs=[_row_spec(IN_DIM), _const_spec(1, 2)],
        out_shape=[
            jax.ShapeDtypeStruct((HALF, IN_DIM), jnp.float32),
            jax.ShapeDtypeStruct((1, 2), jnp.float32),
        ],
        scratch_shapes=[
            pltpu.VMEM((1, 1), jnp.float32),
            pltpu.VMEM((1, 1), jnp.float32),
        ],
    )

    xa, xb = x[:HALF], x[HALF:]

    enc_a, idx0_a = enc_call(xa, g2, be2, W1, b12, W2, b22, cb0)
    enc_b, idx0_b = enc_call(xb, g2, be2, W1, b12, W2, b22, cb0)

    q0_a = _sc_gather(cb0, jnp.reshape(idx0_a, (HALF,)))
    q0_b = _sc_gather(cb0, jnp.reshape(idx0_b, (HALF,)))

    r1_a, idx1_a, c0_a = level_call(enc_a, q0_a, cb1)
    r1_b, idx1_b, c0_b = level_call(enc_b, q0_b, cb1)

    q1_a = _sc_gather(cb1, jnp.reshape(idx1_a, (HALF,)))
    q1_b = _sc_gather(cb1, jnp.reshape(idx1_b, (HALF,)))

    r2_a, idx2_a, c1_a = level_call(r1_a, q1_a, cb2)
    r2_b, idx2_b, c1_b = level_call(r1_b, q1_b, cb2)

    q2_a = _sc_gather(cb2, jnp.reshape(idx2_a, (HALF,)))
    q2_b = _sc_gather(cb2, jnp.reshape(idx2_b, (HALF,)))

    rec_a, p_a = final_call(r2_a, q2_a, enc_a, xa, Wd1, bd12, Wd2, bd22)
    rec_b, p_b = final_call(r2_b, q2_b, enc_b, xb, Wd1, bd12, Wd2, bd22)

    loss = pl.pallas_call(
        _loss_body,
        in_specs=[pl.BlockSpec(memory_space=pltpu.SMEM)] * 4
        + [pl.BlockSpec(memory_space=pltpu.SMEM)] * 2,
        out_specs=pl.BlockSpec(memory_space=pltpu.SMEM),
        out_shape=jax.ShapeDtypeStruct((1, 1), jnp.float32),
    )(c0_a, c0_b, c1_a, c1_b, p_a, p_b)

    rec = jnp.concatenate([rec_a, rec_b], axis=0)
    idx_a = jnp.concatenate([idx0_a, idx1_a, idx2_a], axis=1)
    idx_b = jnp.concatenate([idx0_b, idx1_b, idx2_b], axis=1)
    indices_out = jnp.concatenate([idx_a, idx_b], axis=0)
    return (loss[0, 0], rec, indices_out)


# speedup vs baseline: 1.1210x; 1.1210x over previous
import functools

import jax
import jax.numpy as jnp
from jax import lax
from jax.experimental import pallas as pl
from jax.experimental.pallas import tpu as pltpu
from jax.experimental.pallas import tpu_sc as plsc

B = 16384
IN_DIM = 768
OUT_DIM = 256
K = 1024
BETA = 0.25
EPS = 1e-5

TILE = 1024
GRID = B // TILE


def _argmin_dist(r, cb):
    rn = jnp.sum(r ** 2, axis=1, keepdims=True)
    dot = lax.dot_general(r, cb, (((1,), (1,)), ((), ())))
    cbn = jnp.sum(cb ** 2, axis=1)[None, :]
    d = rn - 2.0 * dot + cbn
    return jnp.argmin(d, axis=1).astype(jnp.int32)


def _enc_body(x_ref, g_ref, be_ref, W1_ref, b1_ref, W2_ref, b2_ref, cb_ref,
              enc_ref, idx_ref):
    x = x_ref[...]
    mu = jnp.mean(x, axis=-1, keepdims=True)
    var = jnp.mean((x - mu) ** 2, axis=-1, keepdims=True)
    xn = (x - mu) / jnp.sqrt(var + EPS) * g_ref[...] + be_ref[...]
    h = jnp.maximum(jnp.dot(xn, W1_ref[...]) + b1_ref[...], 0.0)
    enc = jnp.dot(h, W2_ref[...]) + b2_ref[...]
    enc_ref[...] = enc
    idx_ref[...] = _argmin_dist(enc, cb_ref[...])[:, None]


def _level_body(rprev_ref, q_ref, cb_ref, rout_ref, idx_ref, csum_ref):
    i = pl.program_id(0)
    r = rprev_ref[...] - q_ref[...]
    rout_ref[...] = r
    idx_ref[...] = _argmin_dist(r, cb_ref[...])[:, None]
    s = jnp.sum(r * r).reshape(1, 1)

    @pl.when(i == 0)
    def _():
        csum_ref[...] = s

    @pl.when(i > 0)
    def _():
        csum_ref[...] = csum_ref[...] + s


def _final_body(r2_ref, q2_ref, enc_ref, x_ref, Wd1_ref, bd1_ref, Wd2_ref,
                bd2_ref, c0_ref, c1_ref, rec_ref, loss_ref, accr, accc):
    i = pl.program_id(0)
    r3 = r2_ref[...] - q2_ref[...]
    fq = enc_ref[...] - r3
    dh = jnp.maximum(jnp.dot(fq, Wd1_ref[...]) + bd1_ref[...], 0.0)
    rec = jnp.dot(dh, Wd2_ref[...]) + bd2_ref[...]
    rec_ref[...] = rec
    diff = rec - x_ref[...]
    rs = jnp.sum(diff * diff).reshape(1, 1)
    cs = jnp.sum(r3 * r3).reshape(1, 1)

    @pl.when(i == 0)
    def _():
        accr[...] = rs
        accc[...] = cs

    @pl.when(i > 0)
    def _():
        accr[...] = accr[...] + rs
        accc[...] = accc[...] + cs

    @pl.when(i == GRID - 1)
    def _():
        rl = accr[...] / jnp.float32(B * IN_DIM)
        cl0 = c0_ref[...] / jnp.float32(B * OUT_DIM)
        cl1 = c1_ref[...] / jnp.float32(B * OUT_DIM)
        cl2 = accc[...] / jnp.float32(B * OUT_DIM)
        loss_ref[...] = rl + BETA * (cl0 + cl1 + cl2)


def _row_spec(w):
    return pl.BlockSpec((TILE, w), lambda i: (i, 0))


def _const_spec(h, w):
    return pl.BlockSpec((h, w), lambda i: (0, 0))


_NC = 2
_NS = 16
_NW = _NC * _NS
_ROWS_PER_W = B // _NW
_CHUNK = 256
_NCHUNK = _ROWS_PER_W // _CHUNK


def _sc_gather(table, idx):
    mesh = plsc.VectorSubcoreMesh(core_axis_name="c", subcore_axis_name="s")

    @functools.partial(
        pl.kernel,
        mesh=mesh,
        out_type=jax.ShapeDtypeStruct((B, OUT_DIM), jnp.float32),
        scratch_types=[
            pltpu.VMEM((_CHUNK,), jnp.int32),
            pltpu.VMEM((_CHUNK, OUT_DIM), jnp.float32),
            pltpu.SemaphoreType.DMA,
        ],
    )
    def gk(table_hbm, idx_hbm, out_hbm, idx_v, rows_v, sem):
        wid = lax.axis_index("s") * _NC + lax.axis_index("c")
        for j in range(_NCHUNK):
            base = wid * _ROWS_PER_W + j * _CHUNK
            pltpu.sync_copy(idx_hbm.at[pl.ds(base, _CHUNK)], idx_v)
            pltpu.async_copy(table_hbm.at[idx_v], rows_v, sem).wait()
            pltpu.sync_copy(rows_v, out_hbm.at[pl.ds(base, _CHUNK)])

    return gk(table, idx)


def kernel(x, ln_g, ln_b, W1, b1, W2, b2, Wd1, bd1, Wd2, bd2, codebooks):
    g2 = ln_g[None, :]
    be2 = ln_b[None, :]
    b12 = b1[None, :]
    b22 = b2[None, :]
    bd12 = bd1[None, :]
    bd22 = bd2[None, :]
    cb0, cb1, cb2 = codebooks[0], codebooks[1], codebooks[2]

    enc, idx0 = pl.pallas_call(
        _enc_body,
        grid=(GRID,),
        in_specs=[
            _row_spec(IN_DIM),
            _const_spec(1, IN_DIM),
            _const_spec(1, IN_DIM),
            _const_spec(IN_DIM, OUT_DIM),
            _const_spec(1, OUT_DIM),
            _const_spec(OUT_DIM, OUT_DIM),
            _const_spec(1, OUT_DIM),
            _const_spec(K, OUT_DIM),
        ],
        out_specs=[_row_spec(OUT_DIM), _row_spec(1)],
        out_shape=[
            jax.ShapeDtypeStruct((B, OUT_DIM), jnp.float32),
            jax.ShapeDtypeStruct((B, 1), jnp.int32),
        ],
    )(x, g2, be2, W1, b12, W2, b22, cb0)

    q0 = _sc_gather(cb0, jnp.reshape(idx0, (B,)))

    level = pl.pallas_call(
        _level_body,
        grid=(GRID,),
        in_specs=[_row_spec(OUT_DIM), _row_spec(OUT_DIM), _const_spec(K, OUT_DIM)],
        out_specs=[_row_spec(OUT_DIM), _row_spec(1), _const_spec(1, 1)],
        out_shape=[
            jax.ShapeDtypeStruct((B, OUT_DIM), jnp.float32),
            jax.ShapeDtypeStruct((B, 1), jnp.int32),
            jax.ShapeDtypeStruct((1, 1), jnp.float32),
        ],
    )

    r1, idx1, c0 = level(enc, q0, cb1)
    q1 = _sc_gather(cb1, jnp.reshape(idx1, (B,)))

    r2, idx2, c1 = level(r1, q1, cb2)
    q2 = _sc_gather(cb2, jnp.reshape(idx2, (B,)))

    rec, loss = pl.pallas_call(
        _final_body,
        grid=(GRID,),
        in_specs=[
            _row_spec(OUT_DIM),
            _row_spec(OUT_DIM),
            _row_spec(OUT_DIM),
            _row_spec(IN_DIM),
            _const_spec(OUT_DIM, IN_DIM),
            _const_spec(1, IN_DIM),
            _const_spec(IN_DIM, IN_DIM),
            _const_spec(1, IN_DIM),
            _const_spec(1, 1),
            _const_spec(1, 1),
        ],
        out_specs=[_row_spec(IN_DIM), _const_spec(1, 1)],
        out_shape=[
            jax.ShapeDtypeStruct((B, IN_DIM), jnp.float32),
            jax.ShapeDtypeStruct((1, 1), jnp.float32),
        ],
        scratch_shapes=[
            pltpu.VMEM((1, 1), jnp.float32),
            pltpu.VMEM((1, 1), jnp.float32),
        ],
    )(r2, q2, enc, x, Wd1, bd12, Wd2, bd22, c0, c1)

    indices_out = jnp.concatenate([idx0, idx1, idx2], axis=1)
    return (loss[0, 0], rec, indices_out)
